# baseline (speedup 1.0000x reference)
import jax
import jax.numpy as jnp
from jax.experimental import pallas as pl


def _rownorm_kernel(z_ref, out_ref):
    z = z_ref[...]
    out_ref[...] = -0.5 * jnp.sum(z * z, axis=1)


def kernel(Z, labels):
    del labels
    n, d = Z.shape
    block_rows = 2048
    grid = (n // block_rows,)
    return pl.pallas_call(
        _rownorm_kernel,
        grid=grid,
        in_specs=[pl.BlockSpec((block_rows, d), lambda i: (i, 0))],
        out_specs=pl.BlockSpec((block_rows,), lambda i: (i,)),
        out_shape=jax.ShapeDtypeStruct((n,), Z.dtype),
    )(Z)
